# Initial kernel scaffold; baseline (speedup 1.0000x reference)
import jax
import jax.numpy as jnp
from jax.experimental import pallas as pl


def kernel(gamma, gumbel_noise):
    raise NotImplementedError("write your pallas kernel here")


# speedup vs baseline: 5.1595x; 5.1595x over previous
import jax
import jax.numpy as jnp
from jax.experimental import pallas as pl

_S = 128
_N = 512


def _body(gamma_ref, gn_ref, out_ref):
    scores = gamma_ref[...] + gn_ref[0]
    col = scores.reshape(_N, 1)
    ii = jax.lax.broadcasted_iota(jnp.int32, (_N, _N), 0)
    jj = jax.lax.broadcasted_iota(jnp.int32, (_N, _N), 1)
    lt = col < scores
    eq = col == scores
    c = jnp.logical_or(lt, jnp.logical_and(eq, ii < jj))
    rank = jnp.sum(c.astype(jnp.int32), axis=0, keepdims=True)
    out_ref[0, :, :] = (ii == rank).astype(jnp.float32)


def kernel(gamma, gumbel_noise):
    g2 = gamma.reshape(1, _N)
    gn3 = gumbel_noise.reshape(_S, 1, _N)
    return pl.pallas_call(
        _body,
        grid=(_S,),
        in_specs=[
            pl.BlockSpec((1, _N), lambda s: (0, 0)),
            pl.BlockSpec((1, 1, _N), lambda s: (s, 0, 0)),
        ],
        out_specs=pl.BlockSpec((1, _N, _N), lambda s: (s, 0, 0)),
        out_shape=jax.ShapeDtypeStruct((_S, _N, _N), jnp.float32),
    )(g2, gn3)


# speedup vs baseline: 9.6136x; 1.8633x over previous
import jax
import jax.numpy as jnp
from jax.experimental import pallas as pl

_S = 128
_N = 512
_BS = 4


def _body(gamma_ref, gn_ref, out_ref):
    g = gamma_ref[...]
    gn = gn_ref[0]
    ii = jax.lax.broadcasted_iota(jnp.int32, (_N, _N), 0)
    jj = jax.lax.broadcasted_iota(jnp.int32, (_N, _N), 1)
    tie = ii < jj
    for k in range(_BS):
        scores = g + gn[k:k + 1, :]
        col = scores.reshape(_N, 1)
        c = jnp.logical_or(col < scores,
                           jnp.logical_and(col == scores, tie))
        rank = jnp.sum(c.astype(jnp.int32), axis=0, keepdims=True)
        out_ref[k, :, :] = (ii == rank).astype(jnp.float32)


def kernel(gamma, gumbel_noise):
    g2 = gamma.reshape(1, _N)
    gn3 = gumbel_noise.reshape(_S // _BS, _BS, _N)
    return pl.pallas_call(
        _body,
        grid=(_S // _BS,),
        in_specs=[
            pl.BlockSpec((1, _N), lambda s: (0, 0)),
            pl.BlockSpec((1, _BS, _N), lambda s: (s, 0, 0)),
        ],
        out_specs=pl.BlockSpec((_BS, _N, _N), lambda s: (s, 0, 0)),
        out_shape=jax.ShapeDtypeStruct((_S, _N, _N), jnp.float32),
    )(g2, gn3)


# speedup vs baseline: 10.8116x; 1.1246x over previous
import jax
import jax.numpy as jnp
from jax.experimental import pallas as pl

_S = 128
_N = 512
_BS = 8


def _body(gamma_ref, gn_ref, out_ref):
    g = gamma_ref[...]
    gn = gn_ref[0]
    ii = jax.lax.broadcasted_iota(jnp.int32, (_N, _N), 0)
    jj = jax.lax.broadcasted_iota(jnp.int32, (_N, _N), 1)
    tie = ii < jj
    for k in range(_BS):
        scores = g + gn[k:k + 1, :]
        col = scores.reshape(_N, 1)
        c = jnp.logical_or(col < scores,
                           jnp.logical_and(col == scores, tie))
        rank = jnp.sum(c.astype(jnp.int32), axis=0, keepdims=True)
        out_ref[k, :, :] = (ii == rank).astype(jnp.float32)


def kernel(gamma, gumbel_noise):
    g2 = gamma.reshape(1, _N)
    gn3 = gumbel_noise.reshape(_S // _BS, _BS, _N)
    return pl.pallas_call(
        _body,
        grid=(_S // _BS,),
        in_specs=[
            pl.BlockSpec((1, _N), lambda s: (0, 0)),
            pl.BlockSpec((1, _BS, _N), lambda s: (s, 0, 0)),
        ],
        out_specs=pl.BlockSpec((_BS, _N, _N), lambda s: (s, 0, 0)),
        out_shape=jax.ShapeDtypeStruct((_S, _N, _N), jnp.float32),
    )(g2, gn3)


# speedup vs baseline: 11.3164x; 1.0467x over previous
import jax
import jax.numpy as jnp
from jax.experimental import pallas as pl

_S = 128
_N = 512
_BS = 8

def _body(gamma_ref, gn_ref, out_ref):
    g = gamma_ref[...]
    gn = gn_ref[0]
    ii = jax.lax.broadcasted_iota(jnp.int32, (_N, _N), 0)
    jj = jax.lax.broadcasted_iota(jnp.int32, (_N, _N), 1)
    tie = (ii < jj).astype(jnp.int32)
    ones = jnp.ones((1, _N), jnp.bfloat16)
    for k in range(_BS):
        scores = g + gn[k:k + 1, :]
        bits = jax.lax.bitcast_convert_type(scores, jnp.int32)
        key = jnp.where(bits >= 0, bits,
                        jnp.int32(-2147483648) - bits)
        col = key.reshape(_N, 1)
        c = (col < key + tie).astype(jnp.bfloat16)
        rank = jax.lax.dot_general(
            ones, c, (((1,), (0,)), ((), ())),
            preferred_element_type=jnp.float32)
        out_ref[k, :, :] = (ii == rank.astype(jnp.int32)).astype(jnp.float32)


def kernel(gamma, gumbel_noise):
    g2 = gamma.reshape(1, _N)
    gn3 = gumbel_noise.reshape(_S // _BS, _BS, _N)
    return pl.pallas_call(
        _body,
        grid=(_S // _BS,),
        in_specs=[
            pl.BlockSpec((1, _N), lambda s: (0, 0)),
            pl.BlockSpec((1, _BS, _N), lambda s: (s, 0, 0)),
        ],
        out_specs=pl.BlockSpec((_BS, _N, _N), lambda s: (s, 0, 0)),
        out_shape=jax.ShapeDtypeStruct((_S, _N, _N), jnp.float32),
    )(g2, gn3)
